# baseline (speedup 1.0000x reference)
import functools

import jax
import jax.numpy as jnp
from jax import lax
from jax.experimental import pallas as pl
from jax.experimental.pallas import tpu as pltpu
from jax.experimental.pallas import tpu_sc as plsc

N = 10000
E = 320000
D = 128
NPAD = 10240
NW = 32
K = 80
C = 128
EPW = K * C
EPAD = NW * EPW
HROWS = NPAD // 16
NBUF = 2
K2 = K // 2
BLK = 512


_mesh = plsc.VectorSubcoreMesh(core_axis_name="c", subcore_axis_name="s")


@functools.partial(
    pl.kernel, mesh=_mesh,
    out_type=jax.ShapeDtypeStruct((2, NPAD, D), jnp.float32),
    scratch_types=[
        pltpu.VMEM((K, C), jnp.int32),
        pltpu.VMEM((C, D), jnp.float32),
        pltpu.VMEM_SHARED((NPAD, D), jnp.float32),
    ],
)
def _hist_kernel(dst_hbm, ones_hbm, zeros_hbm, out_hbm, dst_v, ones_v, h_sh):
    c = lax.axis_index("c")
    s = lax.axis_index("s")
    wid = c * 16 + s
    pltpu.sync_copy(dst_hbm.at[wid], dst_v)
    pltpu.sync_copy(ones_hbm, ones_v)
    pltpu.sync_copy(zeros_hbm.at[pl.ds(s * HROWS, HROWS)],
                    h_sh.at[pl.ds(s * HROWS, HROWS)])
    plsc.subcore_barrier()

    def chunk(j, carry):
        pltpu.sync_copy(ones_v, h_sh.at[dst_v.at[j]], add=True)
        return carry

    lax.fori_loop(0, K, chunk, 0)
    plsc.subcore_barrier()
    pltpu.sync_copy(h_sh.at[pl.ds(s * HROWS, HROWS)],
                    out_hbm.at[c, pl.ds(s * HROWS, HROWS)])


@functools.partial(
    pl.kernel, mesh=_mesh,
    out_type=jax.ShapeDtypeStruct((2, NPAD, D), jnp.float32),
    scratch_types=[
        pltpu.VMEM((K, C), jnp.int32),
        pltpu.VMEM((K, C), jnp.int32),
        pltpu.VMEM((C, D), jnp.float32),
        pltpu.VMEM_SHARED((NPAD, D), jnp.float32),
    ],
)
def _edge_scatter(y_hbm, src_hbm, dst_hbm, zeros_hbm, out_hbm,
                  src_v, dst_v, gbuf, z_sh):
    c = lax.axis_index("c")
    s = lax.axis_index("s")
    wid = c * 16 + s
    pltpu.sync_copy(src_hbm.at[wid], src_v)
    pltpu.sync_copy(dst_hbm.at[wid], dst_v)
    pltpu.sync_copy(zeros_hbm.at[pl.ds(s * HROWS, HROWS)],
                    z_sh.at[pl.ds(s * HROWS, HROWS)])
    plsc.subcore_barrier()

    def chunk(j, carry):
        pltpu.sync_copy(y_hbm.at[src_v.at[j]], gbuf)
        pltpu.sync_copy(gbuf, z_sh.at[dst_v.at[j]], add=True)
        return carry

    lax.fori_loop(0, K, chunk, 0)
    plsc.subcore_barrier()
    pltpu.sync_copy(z_sh.at[pl.ds(s * HROWS, HROWS)],
                    out_hbm.at[c, pl.ds(s * HROWS, HROWS)])


def _mm_body(x_ref, w_ref, o_ref):
    o_ref[...] = jnp.dot(x_ref[...], w_ref[...],
                         preferred_element_type=jnp.float32)


def _deg_body(ha_ref, hb_ref, o_ref):
    o_ref[...] = ha_ref[:, :1] + hb_ref[:, :1]


def _scale_body(xw_ref, p_ref, o_ref):
    dis = lax.rsqrt(p_ref[...] + 1.0)
    o_ref[...] = xw_ref[...] * dis


def _comb2_body(za_ref, zb_ref, y1_ref, p_ref, b_ref, w_ref, o_ref):
    dis = lax.rsqrt(p_ref[...] + 1.0)
    pre = (za_ref[...] + zb_ref[...] + y1_ref[...]) * dis + b_ref[...]
    h = jnp.maximum(pre, 0.0)
    rows = lax.broadcasted_iota(jnp.int32, (BLK, 1), 0) + pl.program_id(0) * BLK
    h = jnp.where(rows < N, h, 0.0)
    o_ref[...] = jnp.dot(h, w_ref[...],
                         preferred_element_type=jnp.float32) * dis


def _fin_body(za_ref, zb_ref, y2_ref, p_ref, b_ref, o_ref):
    dis = lax.rsqrt(p_ref[...] + 1.0)
    o_ref[...] = (za_ref[...] + zb_ref[...] + y2_ref[...]) * dis + b_ref[...]


def _row_spec(width=D):
    return pl.BlockSpec((BLK, width), lambda i: (i, 0))


def _full_spec(shape):
    return pl.BlockSpec(shape, lambda i: (0, 0))


def _matmul(x, w):
    return pl.pallas_call(
        _mm_body,
        grid=(NPAD // BLK,),
        in_specs=[_row_spec(), _full_spec((D, D))],
        out_specs=_row_spec(),
        out_shape=jax.ShapeDtypeStruct((NPAD, D), jnp.float32),
    )(x, w)


def _deg(ha, hb):
    return pl.pallas_call(
        _deg_body,
        grid=(NPAD // BLK,),
        in_specs=[_row_spec(), _row_spec()],
        out_specs=_row_spec(1),
        out_shape=jax.ShapeDtypeStruct((NPAD, 1), jnp.float32),
    )(ha, hb)


def _scale(xw, p):
    return pl.pallas_call(
        _scale_body,
        grid=(NPAD // BLK,),
        in_specs=[_row_spec(), _row_spec(1)],
        out_specs=_row_spec(),
        out_shape=jax.ShapeDtypeStruct((NPAD, D), jnp.float32),
    )(xw, p)


def _comb2(za, zb, y1, p, b, w):
    return pl.pallas_call(
        _comb2_body,
        grid=(NPAD // BLK,),
        in_specs=[_row_spec(), _row_spec(), _row_spec(),
                  _row_spec(1),
                  _full_spec((1, D)), _full_spec((D, D))],
        out_specs=_row_spec(),
        out_shape=jax.ShapeDtypeStruct((NPAD, D), jnp.float32),
    )(za, zb, y1, p, b, w)


def _fin(za, zb, y2, p, b):
    return pl.pallas_call(
        _fin_body,
        grid=(NPAD // BLK,),
        in_specs=[_row_spec(), _row_spec(), _row_spec(),
                  _row_spec(1), _full_spec((1, D))],
        out_specs=_row_spec(),
        out_shape=jax.ShapeDtypeStruct((NPAD, D), jnp.float32),
    )(za, zb, y2, p, b)


def kernel(x, edge_index, W1, b1, W2, b2):
    src = edge_index[0].astype(jnp.int32)
    dst = edge_index[1].astype(jnp.int32)
    pad = jnp.full((EPAD - E,), N, jnp.int32)
    srcp = jnp.concatenate([src, pad]).reshape(NW, K, C)
    dstp = jnp.concatenate([dst, pad]).reshape(NW, K, C)
    dsth = dstp
    x_pad = jnp.concatenate(
        [x, jnp.zeros((NPAD - N, D), jnp.float32)], axis=0)
    zeros_z = jnp.zeros((NPAD, D), jnp.float32)
    ones_c = jnp.ones((C, D), jnp.float32)

    hist = _hist_kernel(dsth, ones_c, zeros_z)
    p = _deg(hist[0], hist[1])

    xw1 = _matmul(x_pad, W1)
    y1 = _scale(xw1, p)
    z1 = _edge_scatter(y1, srcp, dstp, zeros_z)
    y2 = _comb2(z1[0], z1[1], y1, p,
                b1.reshape(1, D), W2)
    z2 = _edge_scatter(y2, srcp, dstp, zeros_z)
    out = _fin(z2[0], z2[1], y2, p, b2.reshape(1, D))
    return out[:N]


# speedup vs baseline: 2.1895x; 2.1895x over previous
import functools

import jax
import jax.numpy as jnp
from jax import lax
from jax.experimental import pallas as pl
from jax.experimental.pallas import tpu as pltpu
from jax.experimental.pallas import tpu_sc as plsc

N = 10000
E = 320000
D = 128
NPAD = 10240
NW = 32
K = 80
C = 128
EPW = K * C
EPAD = NW * EPW
HROWS = NPAD // 16
NBUF = 2
K2 = K // 2
BLK = 512


_mesh = plsc.VectorSubcoreMesh(core_axis_name="c", subcore_axis_name="s")


@functools.partial(
    pl.kernel, mesh=_mesh,
    out_type=jax.ShapeDtypeStruct((2, NPAD, D), jnp.float32),
    scratch_types=[
        pltpu.VMEM((K, C), jnp.int32),
        pltpu.VMEM((C, D), jnp.float32),
        pltpu.VMEM_SHARED((NPAD, D), jnp.float32),
    ],
)
def _hist_kernel(dst_hbm, ones_hbm, zeros_hbm, out_hbm, dst_v, ones_v, h_sh):
    c = lax.axis_index("c")
    s = lax.axis_index("s")
    wid = c * 16 + s
    pltpu.sync_copy(dst_hbm.at[wid], dst_v)
    pltpu.sync_copy(ones_hbm, ones_v)
    pltpu.sync_copy(zeros_hbm.at[pl.ds(s * HROWS, HROWS)],
                    h_sh.at[pl.ds(s * HROWS, HROWS)])
    plsc.subcore_barrier()

    def chunk(j, carry):
        pltpu.sync_copy(ones_v, h_sh.at[dst_v.at[j]], add=True)
        return carry

    lax.fori_loop(0, K, chunk, 0)
    plsc.subcore_barrier()
    pltpu.sync_copy(h_sh.at[pl.ds(s * HROWS, HROWS)],
                    out_hbm.at[c, pl.ds(s * HROWS, HROWS)])


@functools.partial(
    pl.kernel, mesh=_mesh,
    out_type=jax.ShapeDtypeStruct((2, NPAD, D), jnp.float32),
    scratch_types=[
        pltpu.VMEM((K, C), jnp.int32),
        pltpu.VMEM((K, C), jnp.int32),
        pltpu.VMEM((C, D), jnp.float32),
        pltpu.VMEM_SHARED((NPAD, D), jnp.float32),
    ],
)
def _edge_scatter(y_hbm, src_hbm, dst_hbm, zeros_hbm, out_hbm,
                  src_v, dst_v, gbuf, z_sh):
    c = lax.axis_index("c")
    s = lax.axis_index("s")
    wid = c * 16 + s
    pltpu.sync_copy(src_hbm.at[wid], src_v)
    pltpu.sync_copy(dst_hbm.at[wid], dst_v)
    pltpu.sync_copy(zeros_hbm.at[pl.ds(s * HROWS, HROWS)],
                    z_sh.at[pl.ds(s * HROWS, HROWS)])
    plsc.subcore_barrier()

    def chunk(j, carry):
        pltpu.sync_copy(y_hbm.at[src_v.at[j]], gbuf)
        pltpu.sync_copy(gbuf, z_sh.at[dst_v.at[j]], add=True)
        return carry

    lax.fori_loop(0, K, chunk, 0)
    plsc.subcore_barrier()
    pltpu.sync_copy(z_sh.at[pl.ds(s * HROWS, HROWS)],
                    out_hbm.at[c, pl.ds(s * HROWS, HROWS)])


def _mm_body(x_ref, w_ref, o_ref):
    o_ref[...] = jnp.dot(x_ref[...], w_ref[...],
                         preferred_element_type=jnp.float32)


def _deg_body(ha_ref, hb_ref, o_ref):
    o_ref[...] = ha_ref[:, :1] + hb_ref[:, :1]


def _scale_body(xw_ref, p_ref, o_ref):
    dis = lax.rsqrt(p_ref[...] + 1.0)
    o_ref[...] = xw_ref[...] * dis


def _comb2_body(za_ref, zb_ref, y1_ref, p_ref, b_ref, w_ref, o_ref):
    dis = lax.rsqrt(p_ref[...] + 1.0)
    pre = (za_ref[...] + zb_ref[...] + y1_ref[...]) * dis + b_ref[...]
    h = jnp.maximum(pre, 0.0)
    rows = lax.broadcasted_iota(jnp.int32, (BLK, 1), 0) + pl.program_id(0) * BLK
    h = jnp.where(rows < N, h, 0.0)
    o_ref[...] = jnp.dot(h, w_ref[...],
                         preferred_element_type=jnp.float32) * dis


def _fin_body(za_ref, zb_ref, y2_ref, p_ref, b_ref, o_ref):
    dis = lax.rsqrt(p_ref[...] + 1.0)
    o_ref[...] = (za_ref[...] + zb_ref[...] + y2_ref[...]) * dis + b_ref[...]


def _row_spec(width=D):
    return pl.BlockSpec((BLK, width), lambda i: (i, 0))


def _full_spec(shape):
    return pl.BlockSpec(shape, lambda i: (0, 0))


def _matmul(x, w):
    return pl.pallas_call(
        _mm_body,
        grid=(NPAD // BLK,),
        in_specs=[_row_spec(), _full_spec((D, D))],
        out_specs=_row_spec(),
        out_shape=jax.ShapeDtypeStruct((NPAD, D), jnp.float32),
    )(x, w)


def _deg(ha, hb):
    return pl.pallas_call(
        _deg_body,
        grid=(NPAD // BLK,),
        in_specs=[_row_spec(), _row_spec()],
        out_specs=_row_spec(1),
        out_shape=jax.ShapeDtypeStruct((NPAD, 1), jnp.float32),
    )(ha, hb)


def _scale(xw, p):
    return pl.pallas_call(
        _scale_body,
        grid=(NPAD // BLK,),
        in_specs=[_row_spec(), _row_spec(1)],
        out_specs=_row_spec(),
        out_shape=jax.ShapeDtypeStruct((NPAD, D), jnp.float32),
    )(xw, p)


def _comb2(za, zb, y1, p, b, w):
    return pl.pallas_call(
        _comb2_body,
        grid=(NPAD // BLK,),
        in_specs=[_row_spec(), _row_spec(), _row_spec(),
                  _row_spec(1),
                  _full_spec((1, D)), _full_spec((D, D))],
        out_specs=_row_spec(),
        out_shape=jax.ShapeDtypeStruct((NPAD, D), jnp.float32),
    )(za, zb, y1, p, b, w)


def _fin(za, zb, y2, p, b):
    return pl.pallas_call(
        _fin_body,
        grid=(NPAD // BLK,),
        in_specs=[_row_spec(), _row_spec(), _row_spec(),
                  _row_spec(1), _full_spec((1, D))],
        out_specs=_row_spec(),
        out_shape=jax.ShapeDtypeStruct((NPAD, D), jnp.float32),
    )(za, zb, y2, p, b)


def kernel(x, edge_index, W1, b1, W2, b2):
    src = edge_index[0].astype(jnp.int32)
    dst = edge_index[1].astype(jnp.int32)
    pad = N + (jnp.arange(EPAD - E, dtype=jnp.int32) % (NPAD - N))
    srcp = jnp.concatenate([src, pad]).reshape(NW, K, C)
    dstp = jnp.concatenate([dst, pad]).reshape(NW, K, C)
    dsth = dstp
    x_pad = jnp.concatenate(
        [x, jnp.zeros((NPAD - N, D), jnp.float32)], axis=0)
    zeros_z = jnp.zeros((NPAD, D), jnp.float32)
    ones_c = jnp.ones((C, D), jnp.float32)

    hist = _hist_kernel(dsth, ones_c, zeros_z)
    p = _deg(hist[0], hist[1])

    xw1 = _matmul(x_pad, W1)
    y1 = _scale(xw1, p)
    z1 = _edge_scatter(y1, srcp, dstp, zeros_z)
    y2 = _comb2(z1[0], z1[1], y1, p,
                b1.reshape(1, D), W2)
    z2 = _edge_scatter(y2, srcp, dstp, zeros_z)
    out = _fin(z2[0], z2[1], y2, p, b2.reshape(1, D))
    return out[:N]


# speedup vs baseline: 2.8788x; 1.3148x over previous
import functools

import jax
import jax.numpy as jnp
from jax import lax
from jax.experimental import pallas as pl
from jax.experimental.pallas import tpu as pltpu
from jax.experimental.pallas import tpu_sc as plsc

N = 10000
E = 320000
D = 128
NPAD = 10240
NW = 32
K = 80
C = 128
EPW = K * C
EPAD = NW * EPW
HROWS = NPAD // 16
NBUF = 2
K2 = K // 2
BLK = 512


_mesh = plsc.VectorSubcoreMesh(core_axis_name="c", subcore_axis_name="s")


@functools.partial(
    pl.kernel, mesh=_mesh,
    out_type=jax.ShapeDtypeStruct((2, NPAD, D), jnp.float32),
    scratch_types=[
        pltpu.VMEM((K, C), jnp.int32),
        pltpu.VMEM((C, D), jnp.float32),
        pltpu.VMEM_SHARED((NPAD, D), jnp.float32),
    ],
)
def _hist_kernel(dst_hbm, ones_hbm, zeros_hbm, out_hbm, dst_v, ones_v, h_sh):
    c = lax.axis_index("c")
    s = lax.axis_index("s")
    wid = c * 16 + s
    pltpu.sync_copy(dst_hbm.at[wid], dst_v)
    pltpu.sync_copy(ones_hbm, ones_v)
    pltpu.sync_copy(zeros_hbm.at[pl.ds(s * HROWS, HROWS)],
                    h_sh.at[pl.ds(s * HROWS, HROWS)])
    plsc.subcore_barrier()

    def chunk(j, carry):
        pltpu.sync_copy(ones_v, h_sh.at[dst_v.at[j]], add=True)
        return carry

    lax.fori_loop(0, K, chunk, 0)
    plsc.subcore_barrier()
    pltpu.sync_copy(h_sh.at[pl.ds(s * HROWS, HROWS)],
                    out_hbm.at[c, pl.ds(s * HROWS, HROWS)])


@functools.partial(
    pl.kernel, mesh=_mesh,
    out_type=jax.ShapeDtypeStruct((2, NPAD, D), jnp.float32),
    scratch_types=[
        pltpu.VMEM((K2, C), jnp.int32),
        pltpu.VMEM((K2, C), jnp.int32),
        [pltpu.VMEM((C, D), jnp.float32)] * NBUF,
        [pltpu.SemaphoreType.DMA] * NBUF,
        pltpu.VMEM_SHARED((NPAD, D), jnp.float32),
    ],
)
def _edge_scatter(y_hbm, src_hbm, dst_hbm, zeros_hbm, out_hbm,
                  src_v, dst_v, gbufs, sems, z_sh):
    c = lax.axis_index("c")
    s = lax.axis_index("s")
    wid = c * 16 + s
    pltpu.sync_copy(zeros_hbm.at[pl.ds(s * HROWS, HROWS)],
                    z_sh.at[pl.ds(s * HROWS, HROWS)])
    plsc.subcore_barrier()

    for h in range(2):
        pltpu.sync_copy(src_hbm.at[wid].at[h], src_v)
        pltpu.sync_copy(dst_hbm.at[wid].at[h], dst_v)
        pltpu.async_copy(y_hbm.at[src_v.at[0]], gbufs[0], sems[0])

        def group(g, carry):
            for b in range(NBUF):
                j = g * NBUF + b
                jn = j + 1

                @pl.when(jn < K2)
                def _():
                    pltpu.async_copy(y_hbm.at[src_v.at[jn]],
                                     gbufs[1 - b], sems[1 - b])

                pltpu.make_async_copy(y_hbm.at[src_v.at[j]], gbufs[b],
                                      sems[b]).wait()
                pltpu.sync_copy(gbufs[b], z_sh.at[dst_v.at[j]], add=True)
            return carry

        lax.fori_loop(0, K2 // NBUF, group, 0)
    plsc.subcore_barrier()
    pltpu.sync_copy(z_sh.at[pl.ds(s * HROWS, HROWS)],
                    out_hbm.at[c, pl.ds(s * HROWS, HROWS)])


def _mm_body(x_ref, w_ref, o_ref):
    o_ref[...] = jnp.dot(x_ref[...], w_ref[...],
                         preferred_element_type=jnp.float32)


def _deg_body(ha_ref, hb_ref, o_ref):
    o_ref[...] = ha_ref[:, :1] + hb_ref[:, :1]


def _scale_body(xw_ref, p_ref, o_ref):
    dis = lax.rsqrt(p_ref[...] + 1.0)
    o_ref[...] = xw_ref[...] * dis


def _comb2_body(za_ref, zb_ref, y1_ref, p_ref, b_ref, w_ref, o_ref):
    dis = lax.rsqrt(p_ref[...] + 1.0)
    pre = (za_ref[...] + zb_ref[...] + y1_ref[...]) * dis + b_ref[...]
    h = jnp.maximum(pre, 0.0)
    rows = lax.broadcasted_iota(jnp.int32, (BLK, 1), 0) + pl.program_id(0) * BLK
    h = jnp.where(rows < N, h, 0.0)
    o_ref[...] = jnp.dot(h, w_ref[...],
                         preferred_element_type=jnp.float32) * dis


def _fin_body(za_ref, zb_ref, y2_ref, p_ref, b_ref, o_ref):
    dis = lax.rsqrt(p_ref[...] + 1.0)
    o_ref[...] = (za_ref[...] + zb_ref[...] + y2_ref[...]) * dis + b_ref[...]


def _row_spec(width=D):
    return pl.BlockSpec((BLK, width), lambda i: (i, 0))


def _full_spec(shape):
    return pl.BlockSpec(shape, lambda i: (0, 0))


def _matmul(x, w):
    return pl.pallas_call(
        _mm_body,
        grid=(NPAD // BLK,),
        in_specs=[_row_spec(), _full_spec((D, D))],
        out_specs=_row_spec(),
        out_shape=jax.ShapeDtypeStruct((NPAD, D), jnp.float32),
    )(x, w)


def _deg(ha, hb):
    return pl.pallas_call(
        _deg_body,
        grid=(NPAD // BLK,),
        in_specs=[_row_spec(), _row_spec()],
        out_specs=_row_spec(1),
        out_shape=jax.ShapeDtypeStruct((NPAD, 1), jnp.float32),
    )(ha, hb)


def _scale(xw, p):
    return pl.pallas_call(
        _scale_body,
        grid=(NPAD // BLK,),
        in_specs=[_row_spec(), _row_spec(1)],
        out_specs=_row_spec(),
        out_shape=jax.ShapeDtypeStruct((NPAD, D), jnp.float32),
    )(xw, p)


def _comb2(za, zb, y1, p, b, w):
    return pl.pallas_call(
        _comb2_body,
        grid=(NPAD // BLK,),
        in_specs=[_row_spec(), _row_spec(), _row_spec(),
                  _row_spec(1),
                  _full_spec((1, D)), _full_spec((D, D))],
        out_specs=_row_spec(),
        out_shape=jax.ShapeDtypeStruct((NPAD, D), jnp.float32),
    )(za, zb, y1, p, b, w)


def _fin(za, zb, y2, p, b):
    return pl.pallas_call(
        _fin_body,
        grid=(NPAD // BLK,),
        in_specs=[_row_spec(), _row_spec(), _row_spec(),
                  _row_spec(1), _full_spec((1, D))],
        out_specs=_row_spec(),
        out_shape=jax.ShapeDtypeStruct((NPAD, D), jnp.float32),
    )(za, zb, y2, p, b)


def kernel(x, edge_index, W1, b1, W2, b2):
    src = edge_index[0].astype(jnp.int32)
    dst = edge_index[1].astype(jnp.int32)
    pad = N + (jnp.arange(EPAD - E, dtype=jnp.int32) % (NPAD - N))
    srcp = jnp.concatenate([src, pad]).reshape(NW, 2, K2, C)
    dstp = jnp.concatenate([dst, pad]).reshape(NW, 2, K2, C)
    dsth = dstp.reshape(NW, K, C)
    x_pad = jnp.concatenate(
        [x, jnp.zeros((NPAD - N, D), jnp.float32)], axis=0)
    zeros_z = jnp.zeros((NPAD, D), jnp.float32)
    ones_c = jnp.ones((C, D), jnp.float32)

    hist = _hist_kernel(dsth, ones_c, zeros_z)
    p = _deg(hist[0], hist[1])

    xw1 = _matmul(x_pad, W1)
    y1 = _scale(xw1, p)
    z1 = _edge_scatter(y1, srcp, dstp, zeros_z)
    y2 = _comb2(z1[0], z1[1], y1, p,
                b1.reshape(1, D), W2)
    z2 = _edge_scatter(y2, srcp, dstp, zeros_z)
    out = _fin(z2[0], z2[1], y2, p, b2.reshape(1, D))
    return out[:N]


# speedup vs baseline: 2.9505x; 1.0249x over previous
import functools

import jax
import jax.numpy as jnp
from jax import lax
from jax.experimental import pallas as pl
from jax.experimental.pallas import tpu as pltpu
from jax.experimental.pallas import tpu_sc as plsc

N = 10000
E = 320000
D = 128
NPAD = 10240
NW = 32
K = 80
C = 128
EPW = K * C
EPAD = NW * EPW
HROWS = NPAD // 16
NBUF = 2
K2 = K // 2
BLK = 512


_mesh = plsc.VectorSubcoreMesh(core_axis_name="c", subcore_axis_name="s")


@functools.partial(
    pl.kernel, mesh=_mesh,
    out_type=jax.ShapeDtypeStruct((2, NPAD, D), jnp.float32),
    scratch_types=[
        pltpu.VMEM((K, C), jnp.int32),
        pltpu.VMEM((C, D), jnp.float32),
        pltpu.VMEM_SHARED((NPAD, D), jnp.float32),
    ],
)
def _hist_kernel(dst_hbm, ones_hbm, zeros_hbm, out_hbm, dst_v, ones_v, h_sh):
    c = lax.axis_index("c")
    s = lax.axis_index("s")
    wid = c * 16 + s
    pltpu.sync_copy(dst_hbm.at[wid], dst_v)
    pltpu.sync_copy(ones_hbm, ones_v)
    pltpu.sync_copy(zeros_hbm.at[pl.ds(s * HROWS, HROWS)],
                    h_sh.at[pl.ds(s * HROWS, HROWS)])
    plsc.subcore_barrier()

    def chunk(j, carry):
        pltpu.sync_copy(ones_v, h_sh.at[dst_v.at[j]], add=True)
        return carry

    lax.fori_loop(0, K, chunk, 0)
    plsc.subcore_barrier()
    pltpu.sync_copy(h_sh.at[pl.ds(s * HROWS, HROWS)],
                    out_hbm.at[c, pl.ds(s * HROWS, HROWS)])


@functools.partial(
    pl.kernel, mesh=_mesh,
    out_type=jax.ShapeDtypeStruct((2, NPAD, D), jnp.float32),
    scratch_types=[
        pltpu.VMEM((K2, C), jnp.int32),
        pltpu.VMEM((K2, C), jnp.int32),
        [pltpu.VMEM((C, D), jnp.float32)] * NBUF,
        [pltpu.SemaphoreType.DMA] * NBUF,
        pltpu.VMEM_SHARED((NPAD, D), jnp.float32),
    ],
)
def _edge_scatter(y_hbm, src_hbm, dst_hbm, zeros_hbm, out_hbm,
                  src_v, dst_v, gbufs, sems, z_sh):
    c = lax.axis_index("c")
    s = lax.axis_index("s")
    wid = c * 16 + s
    pltpu.sync_copy(zeros_hbm.at[pl.ds(s * HROWS, HROWS)],
                    z_sh.at[pl.ds(s * HROWS, HROWS)])
    plsc.subcore_barrier()

    for h in range(2):
        pltpu.sync_copy(src_hbm.at[wid].at[h], src_v)
        pltpu.sync_copy(dst_hbm.at[wid].at[h], dst_v)
        pltpu.async_copy(y_hbm.at[src_v.at[0]], gbufs[0], sems[0])

        def group(g, carry):
            for b in range(NBUF):
                j = g * NBUF + b
                jn = j + 1

                @pl.when(jn < K2)
                def _():
                    pltpu.async_copy(y_hbm.at[src_v.at[jn]],
                                     gbufs[1 - b], sems[1 - b])

                pltpu.make_async_copy(y_hbm.at[src_v.at[j]], gbufs[b],
                                      sems[b]).wait()
                pltpu.sync_copy(gbufs[b], z_sh.at[dst_v.at[j]], add=True)
            return carry

        lax.fori_loop(0, K2 // NBUF, group, 0)
    plsc.subcore_barrier()
    pltpu.sync_copy(z_sh.at[pl.ds(s * HROWS, HROWS)],
                    out_hbm.at[c, pl.ds(s * HROWS, HROWS)])


def _dis(ha_ref, hb_ref):
    return lax.rsqrt(ha_ref[:, :1] + hb_ref[:, :1] + 1.0)


def _mmscale_body(x_ref, w_ref, ha_ref, hb_ref, o_ref):
    o_ref[...] = jnp.dot(x_ref[...], w_ref[...],
                         preferred_element_type=jnp.float32) * _dis(ha_ref,
                                                                    hb_ref)


def _comb2_body(za_ref, zb_ref, y1_ref, ha_ref, hb_ref, b_ref, w_ref, o_ref):
    dis = _dis(ha_ref, hb_ref)
    pre = (za_ref[...] + zb_ref[...] + y1_ref[...]) * dis + b_ref[...]
    h = jnp.maximum(pre, 0.0)
    rows = lax.broadcasted_iota(jnp.int32, (BLK, 1), 0) + pl.program_id(0) * BLK
    h = jnp.where(rows < N, h, 0.0)
    o_ref[...] = jnp.dot(h, w_ref[...],
                         preferred_element_type=jnp.float32) * dis


def _fin_body(za_ref, zb_ref, y2_ref, ha_ref, hb_ref, b_ref, o_ref):
    dis = _dis(ha_ref, hb_ref)
    o_ref[...] = (za_ref[...] + zb_ref[...] + y2_ref[...]) * dis + b_ref[...]


def _row_spec(width=D):
    return pl.BlockSpec((BLK, width), lambda i: (i, 0))


def _full_spec(shape):
    return pl.BlockSpec(shape, lambda i: (0, 0))


def _mmscale(x, w, ha, hb):
    return pl.pallas_call(
        _mmscale_body,
        grid=(NPAD // BLK,),
        in_specs=[_row_spec(), _full_spec((D, D)), _row_spec(), _row_spec()],
        out_specs=_row_spec(),
        out_shape=jax.ShapeDtypeStruct((NPAD, D), jnp.float32),
    )(x, w, ha, hb)


def _comb2(za, zb, y1, ha, hb, b, w):
    return pl.pallas_call(
        _comb2_body,
        grid=(NPAD // BLK,),
        in_specs=[_row_spec(), _row_spec(), _row_spec(),
                  _row_spec(), _row_spec(),
                  _full_spec((1, D)), _full_spec((D, D))],
        out_specs=_row_spec(),
        out_shape=jax.ShapeDtypeStruct((NPAD, D), jnp.float32),
    )(za, zb, y1, ha, hb, b, w)


def _fin(za, zb, y2, ha, hb, b):
    return pl.pallas_call(
        _fin_body,
        grid=(NPAD // BLK,),
        in_specs=[_row_spec(), _row_spec(), _row_spec(),
                  _row_spec(), _row_spec(), _full_spec((1, D))],
        out_specs=_row_spec(),
        out_shape=jax.ShapeDtypeStruct((NPAD, D), jnp.float32),
    )(za, zb, y2, ha, hb, b)


def kernel(x, edge_index, W1, b1, W2, b2):
    src = edge_index[0].astype(jnp.int32)
    dst = edge_index[1].astype(jnp.int32)
    pad = N + (jnp.arange(EPAD - E, dtype=jnp.int32) % (NPAD - N))
    srcp = jnp.concatenate([src, pad]).reshape(NW, 2, K2, C)
    dstp = jnp.concatenate([dst, pad]).reshape(NW, 2, K2, C)
    dsth = dstp.reshape(NW, K, C)
    x_pad = jnp.concatenate(
        [x, jnp.zeros((NPAD - N, D), jnp.float32)], axis=0)
    zeros_z = jnp.zeros((NPAD, D), jnp.float32)
    ones_c = jnp.ones((C, D), jnp.float32)

    hist = _hist_kernel(dsth, ones_c, zeros_z)
    ha, hb = hist[0], hist[1]

    y1 = _mmscale(x_pad, W1, ha, hb)
    z1 = _edge_scatter(y1, srcp, dstp, zeros_z)
    y2 = _comb2(z1[0], z1[1], y1, ha, hb,
                b1.reshape(1, D), W2)
    z2 = _edge_scatter(y2, srcp, dstp, zeros_z)
    out = _fin(z2[0], z2[1], y2, ha, hb, b2.reshape(1, D))
    return out[:N]


# speedup vs baseline: 3.0634x; 1.0383x over previous
import functools

import jax
import jax.numpy as jnp
from jax import lax
from jax.experimental import pallas as pl
from jax.experimental.pallas import tpu as pltpu
from jax.experimental.pallas import tpu_sc as plsc

N = 10000
E = 320000
D = 128
NPAD = 10240
NW = 32
K = 80
C = 128
EPW = K * C
EPAD = NW * EPW
HROWS = NPAD // 16
NBUF = 2
K2 = K // 2
BLK = 1024


_mesh = plsc.VectorSubcoreMesh(core_axis_name="c", subcore_axis_name="s")


@functools.partial(
    pl.kernel, mesh=_mesh,
    out_type=jax.ShapeDtypeStruct((2, NPAD, D), jnp.float32),
    scratch_types=[
        pltpu.VMEM((K, C), jnp.int32),
        pltpu.VMEM((C, D), jnp.float32),
        pltpu.VMEM_SHARED((NPAD, D), jnp.float32),
    ],
)
def _hist_kernel(dst_hbm, ones_hbm, zeros_hbm, out_hbm, dst_v, ones_v, h_sh):
    c = lax.axis_index("c")
    s = lax.axis_index("s")
    wid = c * 16 + s
    pltpu.sync_copy(dst_hbm.at[wid], dst_v)
    pltpu.sync_copy(ones_hbm, ones_v)
    pltpu.sync_copy(zeros_hbm.at[pl.ds(s * HROWS, HROWS)],
                    h_sh.at[pl.ds(s * HROWS, HROWS)])
    plsc.subcore_barrier()

    def chunk(j, carry):
        pltpu.sync_copy(ones_v, h_sh.at[dst_v.at[j]], add=True)
        return carry

    lax.fori_loop(0, K, chunk, 0)
    plsc.subcore_barrier()
    pltpu.sync_copy(h_sh.at[pl.ds(s * HROWS, HROWS)],
                    out_hbm.at[c, pl.ds(s * HROWS, HROWS)])


@functools.partial(
    pl.kernel, mesh=_mesh,
    out_type=jax.ShapeDtypeStruct((2, NPAD, D), jnp.float32),
    scratch_types=[
        pltpu.VMEM((K2, C), jnp.int32),
        pltpu.VMEM((K2, C), jnp.int32),
        [pltpu.VMEM((C, D), jnp.float32)] * NBUF,
        [pltpu.SemaphoreType.DMA] * NBUF,
        pltpu.VMEM_SHARED((NPAD, D), jnp.float32),
    ],
)
def _edge_scatter(y_hbm, src_hbm, dst_hbm, zeros_hbm, out_hbm,
                  src_v, dst_v, gbufs, sems, z_sh):
    c = lax.axis_index("c")
    s = lax.axis_index("s")
    wid = c * 16 + s
    pltpu.sync_copy(zeros_hbm.at[pl.ds(s * HROWS, HROWS)],
                    z_sh.at[pl.ds(s * HROWS, HROWS)])
    plsc.subcore_barrier()

    for h in range(2):
        pltpu.sync_copy(src_hbm.at[wid].at[h], src_v)
        pltpu.sync_copy(dst_hbm.at[wid].at[h], dst_v)
        pltpu.async_copy(y_hbm.at[src_v.at[0]], gbufs[0], sems[0])

        def group(g, carry):
            for b in range(NBUF):
                j = g * NBUF + b
                jn = j + 1

                @pl.when(jn < K2)
                def _():
                    pltpu.async_copy(y_hbm.at[src_v.at[jn]],
                                     gbufs[1 - b], sems[1 - b])

                pltpu.make_async_copy(y_hbm.at[src_v.at[j]], gbufs[b],
                                      sems[b]).wait()
                pltpu.sync_copy(gbufs[b], z_sh.at[dst_v.at[j]], add=True)
            return carry

        lax.fori_loop(0, K2 // NBUF, group, 0)
    plsc.subcore_barrier()
    pltpu.sync_copy(z_sh.at[pl.ds(s * HROWS, HROWS)],
                    out_hbm.at[c, pl.ds(s * HROWS, HROWS)])


def _dis(ha_ref, hb_ref):
    return lax.rsqrt(ha_ref[:, :1] + hb_ref[:, :1] + 1.0)


def _mmscale_body(x_ref, w_ref, ha_ref, hb_ref, o_ref):
    o_ref[...] = jnp.dot(x_ref[...], w_ref[...],
                         preferred_element_type=jnp.float32) * _dis(ha_ref,
                                                                    hb_ref)


def _comb2_body(za_ref, zb_ref, y1_ref, ha_ref, hb_ref, b_ref, w_ref, o_ref):
    dis = _dis(ha_ref, hb_ref)
    pre = (za_ref[...] + zb_ref[...] + y1_ref[...]) * dis + b_ref[...]
    h = jnp.maximum(pre, 0.0)
    rows = lax.broadcasted_iota(jnp.int32, (BLK, 1), 0) + pl.program_id(0) * BLK
    h = jnp.where(rows < N, h, 0.0)
    o_ref[...] = jnp.dot(h, w_ref[...],
                         preferred_element_type=jnp.float32) * dis


def _fin_body(za_ref, zb_ref, y2_ref, ha_ref, hb_ref, b_ref, o_ref):
    dis = _dis(ha_ref, hb_ref)
    o_ref[...] = (za_ref[...] + zb_ref[...] + y2_ref[...]) * dis + b_ref[...]


def _row_spec(width=D):
    return pl.BlockSpec((BLK, width), lambda i: (i, 0))


def _full_spec(shape):
    return pl.BlockSpec(shape, lambda i: (0, 0))


def _mmscale(x, w, ha, hb):
    return pl.pallas_call(
        _mmscale_body,
        grid=(NPAD // BLK,),
        in_specs=[_row_spec(), _full_spec((D, D)), _row_spec(), _row_spec()],
        out_specs=_row_spec(),
        out_shape=jax.ShapeDtypeStruct((NPAD, D), jnp.float32),
    )(x, w, ha, hb)


def _comb2(za, zb, y1, ha, hb, b, w):
    return pl.pallas_call(
        _comb2_body,
        grid=(NPAD // BLK,),
        in_specs=[_row_spec(), _row_spec(), _row_spec(),
                  _row_spec(), _row_spec(),
                  _full_spec((1, D)), _full_spec((D, D))],
        out_specs=_row_spec(),
        out_shape=jax.ShapeDtypeStruct((NPAD, D), jnp.float32),
    )(za, zb, y1, ha, hb, b, w)


def _fin(za, zb, y2, ha, hb, b):
    return pl.pallas_call(
        _fin_body,
        grid=(NPAD // BLK,),
        in_specs=[_row_spec(), _row_spec(), _row_spec(),
                  _row_spec(), _row_spec(), _full_spec((1, D))],
        out_specs=_row_spec(),
        out_shape=jax.ShapeDtypeStruct((NPAD, D), jnp.float32),
    )(za, zb, y2, ha, hb, b)


def kernel(x, edge_index, W1, b1, W2, b2):
    src = edge_index[0].astype(jnp.int32)
    dst = edge_index[1].astype(jnp.int32)
    pad = N + (jnp.arange(EPAD - E, dtype=jnp.int32) % (NPAD - N))
    srcp = jnp.concatenate([src, pad]).reshape(NW, 2, K2, C)
    dstp = jnp.concatenate([dst, pad]).reshape(NW, 2, K2, C)
    dsth = dstp.reshape(NW, K, C)
    x_pad = jnp.concatenate(
        [x, jnp.zeros((NPAD - N, D), jnp.float32)], axis=0)
    zeros_z = jnp.zeros((NPAD, D), jnp.float32)
    ones_c = jnp.ones((C, D), jnp.float32)

    hist = _hist_kernel(dsth, ones_c, zeros_z)
    ha, hb = hist[0], hist[1]

    y1 = _mmscale(x_pad, W1, ha, hb)
    z1 = _edge_scatter(y1, srcp, dstp, zeros_z)
    y2 = _comb2(z1[0], z1[1], y1, ha, hb,
                b1.reshape(1, D), W2)
    z2 = _edge_scatter(y2, srcp, dstp, zeros_z)
    out = _fin(z2[0], z2[1], y2, ha, hb, b2.reshape(1, D))
    return out[:N]
